# baseline (speedup 1.0000x reference)
import jax
import jax.numpy as jnp
from jax.experimental import pallas as pl
from jax.experimental.pallas import tpu as pltpu

_TEMPERATURE = 45.254834
_LOG2E_OVER_T = 1.4426950408889634 / _TEMPERATURE

_KCHUNK = 512
_W = 512


def _stage_body(q_ref, kb_ref, o_ref, dbuf, dm, ebuf, em, er, sbuf, sr):
    nk = kb_ref.shape[0]
    qs = (q_ref[...] * _LOG2E_OVER_T).astype(jnp.bfloat16)
    mval = em[...]
    rval = sr[...]
    macc = None
    ssum = None
    for c in range(nk // _W):
        sl = pl.ds(c * _W, _W)
        x_c = jax.lax.dot_general(
            qs,
            kb_ref[sl, :],
            (((1,), (1,)), ((), ())),
            preferred_element_type=jnp.float32,
        )
        dbuf[:, sl] = x_c
        m_c = jnp.max(x_c, axis=-1, keepdims=True)
        macc = m_c if macc is None else jnp.maximum(macc, m_c)
        e_c = jnp.exp2(ebuf[:, sl] - mval)
        ebuf[:, sl] = e_c
        s_c = jnp.sum(e_c, axis=-1, keepdims=True)
        ssum = s_c if ssum is None else ssum + s_c
        o_ref[:, sl] = sbuf[:, sl] * rval
    dm[...] = macc
    er[...] = 1.0 / ssum


def _attn_kernel(q_ref, k_ref, o_ref,
                 x0, x1, x2, m0, m1, m2, r0, r1, r2,
                 kb_ref, kf_ref, sem):
    i = pl.program_id(0)
    nk = kb_ref.shape[0]

    @pl.when(i == 0)
    def _load_k():
        def copy(c, buf):
            return pltpu.make_async_copy(
                k_ref.at[pl.ds(c * _KCHUNK, _KCHUNK), :],
                kf_ref.at[buf],
                sem.at[buf],
            )

        copy(0, 0).start()
        for c in range(nk // _KCHUNK):
            if c + 1 < nk // _KCHUNK:
                copy(c + 1, (c + 1) % 2).start()
            copy(c, c % 2).wait()
            kb_ref[pl.ds(c * _KCHUNK, _KCHUNK), :] = (
                kf_ref[c % 2].astype(jnp.bfloat16)
            )

    p = jax.lax.rem(i, 3)

    @pl.when(p == 0)
    def _p0():
        _stage_body(q_ref, kb_ref, o_ref, x0, m0, x2, m2, r2, x1, r1)

    @pl.when(p == 1)
    def _p1():
        _stage_body(q_ref, kb_ref, o_ref, x1, m1, x0, m0, r0, x2, r2)

    @pl.when(p == 2)
    def _p2():
        _stage_body(q_ref, kb_ref, o_ref, x2, m2, x1, m1, r1, x0, r0)


def kernel(q, k):
    n, d = q.shape
    nk = k.shape[0]
    br = 256
    ni = n // br
    stripe = lambda: pltpu.VMEM((br, nk), jnp.float32)
    vec = lambda: pltpu.VMEM((br, 1), jnp.float32)
    return pl.pallas_call(
        _attn_kernel,
        grid=(ni + 2,),
        in_specs=[
            pl.BlockSpec((br, d), lambda i: (jnp.minimum(i, ni - 1), 0)),
            pl.BlockSpec(memory_space=pltpu.MemorySpace.HBM),
        ],
        out_specs=pl.BlockSpec(
            (br, nk), lambda i: (jnp.maximum(i - 2, 0), 0)
        ),
        out_shape=jax.ShapeDtypeStruct((n, nk), jnp.float32),
        scratch_shapes=[
            stripe(), stripe(), stripe(),
            vec(), vec(), vec(), vec(), vec(), vec(),
            pltpu.VMEM((nk, d), jnp.bfloat16),
            pltpu.VMEM((2, _KCHUNK, d), jnp.float32),
            pltpu.SemaphoreType.DMA((2,)),
        ],
        compiler_params=pltpu.CompilerParams(
            dimension_semantics=("arbitrary",)
        ),
    )(q, k)
